# baseline (speedup 1.0000x reference)
import functools

import jax
import jax.numpy as jnp
from jax import lax
from jax.experimental import pallas as pl
from jax.experimental.pallas import tpu as pltpu
from jax.experimental.pallas import tpu_sc as plsc

B, T, V, D, H = 2048, 32, 100, 128, 256
BT = B * T
CH = 256


def _sc_gather(table, idx_flat):
    info = plsc.get_sparse_core_info()
    nw = info.num_cores * info.num_subcores
    b_per_w = BT // nw
    mesh = plsc.VectorSubcoreMesh(core_axis_name="c", subcore_axis_name="s")

    @functools.partial(
        pl.kernel, mesh=mesh,
        out_type=jax.ShapeDtypeStruct((BT, D), jnp.float32),
        scratch_types=[
            pltpu.VMEM((CH,), jnp.int32),
            pltpu.VMEM((CH, D), jnp.float32),
            pltpu.SemaphoreType.DMA,
        ],
    )
    def k(table_hbm, idx_hbm, out_hbm, idx_v, rows_v, sem):
        wid = lax.axis_index("s") * info.num_cores + lax.axis_index("c")
        base = wid * b_per_w
        for c in range(b_per_w // CH):
            off = base + c * CH
            pltpu.sync_copy(idx_hbm.at[pl.ds(off, CH)], idx_v)
            pltpu.async_copy(table_hbm.at[idx_v], rows_v, sem).wait()
            pltpu.sync_copy(rows_v, out_hbm.at[pl.ds(off, CH)])

    return k(table, idx_flat)


def _gru_kernel(x_ref, h0_ref, len_ref, wihT_ref, whha_ref, bihn_ref, out_ref):
    wihb = wihT_ref[...].astype(jnp.bfloat16)
    whhab = whha_ref[...].astype(jnp.bfloat16)
    bih_n = bihn_ref[...]
    lens = len_ref[...]
    h = h0_ref[...]
    BBl = h.shape[0]
    ones = jnp.ones((BBl, 1), jnp.bfloat16)

    for t in range(T):
        xt = x_ref[:, t * D:(t + 1) * D]
        gi = jnp.dot(xt.astype(jnp.bfloat16), wihb,
                     preferred_element_type=jnp.float32)
        hb1 = jnp.concatenate([h.astype(jnp.bfloat16), ones], axis=1)
        gh = jnp.dot(hb1, whhab, preferred_element_type=jnp.float32)
        r = jax.nn.sigmoid(gi[:, :H] + gh[:, :H])
        z = jax.nn.sigmoid(gi[:, H:2 * H] + gh[:, H:2 * H])
        n = jnp.tanh(gi[:, 2 * H:] + bih_n + r * gh[:, 2 * H:])
        h = n + z * (h - n)
        out_ref[:, t, :] = jnp.where(t < lens, h, 0.0)


@jax.jit
def _run(x, h0, lens2d, wihT, whha, bihn):
    BB = 256
    grid = (B // BB,)
    return pl.pallas_call(
        _gru_kernel,
        grid=grid,
        in_specs=[
            pl.BlockSpec((BB, T * D), lambda i: (i, 0)),
            pl.BlockSpec((BB, H), lambda i: (i, 0)),
            pl.BlockSpec((BB, 1), lambda i: (i, 0)),
            pl.BlockSpec((D, 3 * H), lambda i: (0, 0)),
            pl.BlockSpec((H + 1, 3 * H), lambda i: (0, 0)),
            pl.BlockSpec((1, H), lambda i: (0, 0)),
        ],
        out_specs=pl.BlockSpec((BB, T, H), lambda i: (i, 0, 0)),
        out_shape=jax.ShapeDtypeStruct((B, T, H), jnp.float32),
        compiler_params=pltpu.CompilerParams(
            dimension_semantics=("parallel",)),
    )(x, h0, lens2d, wihT, whha, bihn)


def kernel(output, conditioning, output_mask, output_word_len, emb,
           W_ih, W_hh, b_ih, b_hh):
    h0 = conditioning[0]
    lens2d = jnp.maximum(output_word_len, 1)[:, None].astype(jnp.int32)
    x = _sc_gather(emb, output.reshape(BT).astype(jnp.int32))
    x = x.reshape(B, T * D)
    bias_row = jnp.concatenate(
        [(b_ih + b_hh)[:2 * H], b_hh[2 * H:]])[None, :]
    whha = jnp.concatenate([W_hh.T, bias_row], axis=0)
    return _run(x, h0, lens2d, W_ih.T, whha, b_ih[None, 2 * H:])


# speedup vs baseline: 2.6585x; 2.6585x over previous
import functools

import jax
import jax.numpy as jnp
from jax.experimental import pallas as pl
from jax.experimental.pallas import tpu as pltpu

B, T, V, D, H = 2048, 32, 100, 128, 256


def _gru_kernel(idx_ref, h0_ref, len_ref, emb_ref, wihT_ref, whhT_ref,
                bih_ref, bhh_ref, out_ref):
    bias = bih_ref[...] + jnp.concatenate(
        [bhh_ref[:, :2 * H], jnp.zeros((1, H), jnp.float32)], axis=1)
    G = jnp.dot(emb_ref[...].astype(jnp.bfloat16), wihT_ref[...].astype(jnp.bfloat16),
                preferred_element_type=jnp.float32) + bias
    Gb = G.astype(jnp.bfloat16)
    whhT = whhT_ref[...].astype(jnp.bfloat16)
    bhh_n = bhh_ref[0, 2 * H:][None, :]
    lens = len_ref[...]
    idx = idx_ref[...]
    h = h0_ref[...]
    iota_v = jax.lax.broadcasted_iota(jnp.int32, (1, V), 1)

    for t in range(T):
        onehot = (idx[:, t][:, None] == iota_v).astype(jnp.bfloat16)
        gi = jnp.dot(onehot, Gb, preferred_element_type=jnp.float32)
        gh = jnp.dot(h.astype(jnp.bfloat16), whhT,
                     preferred_element_type=jnp.float32)
        r = jax.nn.sigmoid(gi[:, :H] + gh[:, :H])
        z = jax.nn.sigmoid(gi[:, H:2 * H] + gh[:, H:2 * H])
        n = jnp.tanh(gi[:, 2 * H:] + r * (gh[:, 2 * H:] + bhh_n))
        h = n + z * (h - n)
        out_ref[:, t, :] = jnp.where(t < lens, h, 0.0)


@functools.partial(jax.jit, static_argnames=("interpret",))
def _run(output, h0, lens2d, emb, wihT, whhT, bih2d, bhh2d, interpret=False):
    BB = 256
    grid = (B // BB,)
    return pl.pallas_call(
        _gru_kernel,
        grid=grid,
        in_specs=[
            pl.BlockSpec((BB, T), lambda i: (i, 0)),
            pl.BlockSpec((BB, H), lambda i: (i, 0)),
            pl.BlockSpec((BB, 1), lambda i: (i, 0)),
            pl.BlockSpec((V, D), lambda i: (0, 0)),
            pl.BlockSpec((D, 3 * H), lambda i: (0, 0)),
            pl.BlockSpec((H, 3 * H), lambda i: (0, 0)),
            pl.BlockSpec((1, 3 * H), lambda i: (0, 0)),
            pl.BlockSpec((1, 3 * H), lambda i: (0, 0)),
        ],
        out_specs=pl.BlockSpec((BB, T, H), lambda i: (i, 0, 0)),
        out_shape=jax.ShapeDtypeStruct((B, T, H), jnp.float32),
        compiler_params=pltpu.CompilerParams(
            dimension_semantics=("parallel",)),
        interpret=interpret,
    )(output, h0, lens2d, emb, wihT, whhT, bih2d, bhh2d)


def kernel(output, conditioning, output_mask, output_word_len, emb,
           W_ih, W_hh, b_ih, b_hh, interpret=False):
    h0 = conditioning[0]
    lens2d = jnp.maximum(output_word_len, 1)[:, None].astype(jnp.int32)
    return _run(output.astype(jnp.int32), h0, lens2d, emb,
                W_ih.T, W_hh.T, b_ih[None, :], b_hh[None, :],
                interpret=interpret)


# speedup vs baseline: 2.6596x; 1.0004x over previous
import jax
import jax.numpy as jnp
from jax.experimental import pallas as pl
from jax.experimental.pallas import tpu as pltpu

B, T, V, D, H = 2048, 32, 100, 128, 256


def _gru_kernel(idx_ref, h0_ref, len_ref, emb_ref, wihT_ref, whhT_ref,
                bih_ref, bhh_ref, out_ref):
    bias = bih_ref[...] + jnp.concatenate(
        [bhh_ref[:, :2 * H], jnp.zeros((1, H), jnp.float32)], axis=1)
    G = jnp.dot(emb_ref[...].astype(jnp.bfloat16), wihT_ref[...].astype(jnp.bfloat16),
                preferred_element_type=jnp.float32) + bias
    Gb = G.astype(jnp.bfloat16)
    whhT = whhT_ref[...].astype(jnp.bfloat16)
    bhh_n = bhh_ref[0, 2 * H:][None, :]
    lens = len_ref[...]
    idx = idx_ref[...]
    h = h0_ref[...]
    iota_v = jax.lax.broadcasted_iota(jnp.int32, (1, V), 1)

    for t in range(T):
        onehot = (idx[:, t][:, None] == iota_v).astype(jnp.bfloat16)
        gi = jnp.dot(onehot, Gb, preferred_element_type=jnp.float32)
        gh = jnp.dot(h.astype(jnp.bfloat16), whhT,
                     preferred_element_type=jnp.float32)
        r = jax.nn.sigmoid(gi[:, :H] + gh[:, :H])
        z = jax.nn.sigmoid(gi[:, H:2 * H] + gh[:, H:2 * H])
        n = jnp.tanh(gi[:, 2 * H:] + r * (gh[:, 2 * H:] + bhh_n))
        h = n + z * (h - n)
        out_ref[:, t, :] = jnp.where(t < lens, h, 0.0)


@jax.jit
def _run(output, h0, lens2d, emb, wihT, whhT, bih2d, bhh2d):
    BB = 256
    grid = (B // BB,)
    return pl.pallas_call(
        _gru_kernel,
        grid=grid,
        in_specs=[
            pl.BlockSpec((BB, T), lambda i: (i, 0)),
            pl.BlockSpec((BB, H), lambda i: (i, 0)),
            pl.BlockSpec((BB, 1), lambda i: (i, 0)),
            pl.BlockSpec((V, D), lambda i: (0, 0)),
            pl.BlockSpec((D, 3 * H), lambda i: (0, 0)),
            pl.BlockSpec((H, 3 * H), lambda i: (0, 0)),
            pl.BlockSpec((1, 3 * H), lambda i: (0, 0)),
            pl.BlockSpec((1, 3 * H), lambda i: (0, 0)),
        ],
        out_specs=pl.BlockSpec((BB, T, H), lambda i: (i, 0, 0)),
        out_shape=jax.ShapeDtypeStruct((B, T, H), jnp.float32),
        compiler_params=pltpu.CompilerParams(
            dimension_semantics=("parallel",)),
    )(output, h0, lens2d, emb, wihT, whhT, bih2d, bhh2d)


def kernel(output, conditioning, output_mask, output_word_len, emb,
           W_ih, W_hh, b_ih, b_hh):
    h0 = conditioning[0]
    lens2d = jnp.maximum(output_word_len, 1)[:, None].astype(jnp.int32)
    return _run(output.astype(jnp.int32), h0, lens2d, emb,
                W_ih.T, W_hh.T, b_ih[None, :], b_hh[None, :])
